# baseline (device time: 36632 ns/iter reference)
import jax
import jax.numpy as jnp
from jax import lax
from jax.experimental import pallas as pl
from jax.experimental.pallas import tpu as pltpu

N_Y = 4
M = 512
D = 512
CH = M // N_Y


def kernel(partial, resid, gamma):
    x = partial.reshape(M, D)
    g = gamma.reshape(1, D)

    def body(x_ref, resid_ref, g_ref, out_ref, send_buf, recv_buf,
             rs_send_sems, rs_recv_sems, ag_send_sems, ag_recv_sems):
        mx = lax.axis_index("x")
        my = lax.axis_index("y")
        mz = lax.axis_index("z")
        right = (mx, (my + 1) % N_Y, mz)
        left = (mx, (my - 1) % N_Y, mz)

        barrier_sem = pltpu.get_barrier_semaphore()
        for nbr in (left, right):
            pl.semaphore_signal(
                barrier_sem, inc=1,
                device_id=nbr, device_id_type=pl.DeviceIdType.MESH,
            )
        pl.semaphore_wait(barrier_sem, 2)

        send_buf[0] = x_ref[pl.ds(my * CH, CH), :]
        acc = None
        for h in range(N_Y - 1):
            rdma = pltpu.make_async_remote_copy(
                src_ref=send_buf.at[h],
                dst_ref=recv_buf.at[h],
                send_sem=rs_send_sems.at[h],
                recv_sem=rs_recv_sems.at[h],
                device_id=right,
                device_id_type=pl.DeviceIdType.MESH,
            )
            rdma.start()
            rdma.wait()
            c = (my - h - 1) % N_Y
            chunk = recv_buf[h] + x_ref[pl.ds(c * CH, CH), :]
            if h < N_Y - 2:
                send_buf[h + 1] = chunk
            else:
                acc = chunk

        own = (my + 1) % N_Y
        rows = pl.ds(own * CH, CH)
        y = acc + resid_ref[rows, :]
        rms = jnp.sqrt(jnp.mean(y * y, axis=-1, keepdims=True) + 1e-6)
        out_ref[rows, :] = y / rms * g_ref[:, :]

        for t in range(N_Y - 1):
            sc = (my + 1 - t) % N_Y
            rdma = pltpu.make_async_remote_copy(
                src_ref=out_ref.at[pl.ds(sc * CH, CH), :],
                dst_ref=out_ref.at[pl.ds(sc * CH, CH), :],
                send_sem=ag_send_sems.at[t],
                recv_sem=ag_recv_sems.at[t],
                device_id=right,
                device_id_type=pl.DeviceIdType.MESH,
            )
            rdma.start()
            rdma.wait()

    return pl.pallas_call(
        body,
        out_shape=jax.ShapeDtypeStruct((M, D), jnp.float32),
        in_specs=[
            pl.BlockSpec(memory_space=pltpu.VMEM),
            pl.BlockSpec(memory_space=pltpu.VMEM),
            pl.BlockSpec(memory_space=pltpu.VMEM),
        ],
        out_specs=pl.BlockSpec(memory_space=pltpu.VMEM),
        scratch_shapes=[
            pltpu.VMEM((N_Y - 1, CH, D), jnp.float32),
            pltpu.VMEM((N_Y - 1, CH, D), jnp.float32),
            pltpu.SemaphoreType.DMA((N_Y - 1,)),
            pltpu.SemaphoreType.DMA((N_Y - 1,)),
            pltpu.SemaphoreType.DMA((N_Y - 1,)),
            pltpu.SemaphoreType.DMA((N_Y - 1,)),
        ],
        compiler_params=pltpu.CompilerParams(collective_id=0),
    )(x, resid, g)


# device time: 34811 ns/iter; 1.0523x vs baseline; 1.0523x over previous
import jax
import jax.numpy as jnp
from jax import lax
from jax.experimental import pallas as pl
from jax.experimental.pallas import tpu as pltpu

N_Y = 4
M = 512
D = 512
CH = M // (2 * N_Y)
B_OFF = M // 2


def kernel(partial, resid, gamma):
    x = partial.reshape(M, D)
    g = gamma.reshape(1, D)

    def body(x_ref, resid_ref, g_ref, out_ref,
             send_a, recv_a, send_b, recv_b,
             rs_ss_a, rs_rs_a, rs_ss_b, rs_rs_b,
             ag_ss_a, ag_rs_a, ag_ss_b, ag_rs_b):
        mx = lax.axis_index("x")
        my = lax.axis_index("y")
        mz = lax.axis_index("z")
        right = (mx, (my + 1) % N_Y, mz)
        left = (mx, (my - 1) % N_Y, mz)

        barrier_sem = pltpu.get_barrier_semaphore()
        for nbr in (left, right):
            pl.semaphore_signal(
                barrier_sem, inc=1,
                device_id=nbr, device_id_type=pl.DeviceIdType.MESH,
            )
        pl.semaphore_wait(barrier_sem, 2)

        def a_rows(c):
            return pl.ds(c * CH, CH)

        def b_rows(c):
            return pl.ds(B_OFF + c * CH, CH)

        send_a[0] = x_ref[a_rows(my), :]
        send_b[0] = x_ref[b_rows(my), :]
        acc_a = acc_b = None
        for h in range(N_Y - 1):
            rdma_a = pltpu.make_async_remote_copy(
                src_ref=send_a.at[h], dst_ref=recv_a.at[h],
                send_sem=rs_ss_a.at[h], recv_sem=rs_rs_a.at[h],
                device_id=right, device_id_type=pl.DeviceIdType.MESH,
            )
            rdma_b = pltpu.make_async_remote_copy(
                src_ref=send_b.at[h], dst_ref=recv_b.at[h],
                send_sem=rs_ss_b.at[h], recv_sem=rs_rs_b.at[h],
                device_id=left, device_id_type=pl.DeviceIdType.MESH,
            )
            rdma_a.start()
            rdma_b.start()
            rdma_a.wait()
            rdma_b.wait()
            ca = (my - h - 1) % N_Y
            cb = (my + h + 1) % N_Y
            chunk_a = recv_a[h] + x_ref[a_rows(ca), :]
            chunk_b = recv_b[h] + x_ref[b_rows(cb), :]
            if h < N_Y - 2:
                send_a[h + 1] = chunk_a
                send_b[h + 1] = chunk_b
            else:
                acc_a = chunk_a
                acc_b = chunk_b

        own_a = (my + 1) % N_Y
        own_b = (my - 1) % N_Y
        ya = acc_a + resid_ref[a_rows(own_a), :]
        rms_a = jnp.sqrt(jnp.mean(ya * ya, axis=-1, keepdims=True) + 1e-6)
        out_ref[a_rows(own_a), :] = ya / rms_a * g_ref[:, :]
        yb = acc_b + resid_ref[b_rows(own_b), :]
        rms_b = jnp.sqrt(jnp.mean(yb * yb, axis=-1, keepdims=True) + 1e-6)
        out_ref[b_rows(own_b), :] = yb / rms_b * g_ref[:, :]

        for t in range(N_Y - 1):
            sa = (my + 1 - t) % N_Y
            sb = (my - 1 + t) % N_Y
            rdma_a = pltpu.make_async_remote_copy(
                src_ref=out_ref.at[a_rows(sa), :],
                dst_ref=out_ref.at[a_rows(sa), :],
                send_sem=ag_ss_a.at[t], recv_sem=ag_rs_a.at[t],
                device_id=right, device_id_type=pl.DeviceIdType.MESH,
            )
            rdma_b = pltpu.make_async_remote_copy(
                src_ref=out_ref.at[b_rows(sb), :],
                dst_ref=out_ref.at[b_rows(sb), :],
                send_sem=ag_ss_b.at[t], recv_sem=ag_rs_b.at[t],
                device_id=left, device_id_type=pl.DeviceIdType.MESH,
            )
            rdma_a.start()
            rdma_b.start()
            rdma_a.wait()
            rdma_b.wait()

    n_h = N_Y - 1
    return pl.pallas_call(
        body,
        out_shape=jax.ShapeDtypeStruct((M, D), jnp.float32),
        in_specs=[
            pl.BlockSpec(memory_space=pltpu.VMEM),
            pl.BlockSpec(memory_space=pltpu.VMEM),
            pl.BlockSpec(memory_space=pltpu.VMEM),
        ],
        out_specs=pl.BlockSpec(memory_space=pltpu.VMEM),
        scratch_shapes=[
            pltpu.VMEM((n_h, CH, D), jnp.float32),
            pltpu.VMEM((n_h, CH, D), jnp.float32),
            pltpu.VMEM((n_h, CH, D), jnp.float32),
            pltpu.VMEM((n_h, CH, D), jnp.float32),
            pltpu.SemaphoreType.DMA((n_h,)),
            pltpu.SemaphoreType.DMA((n_h,)),
            pltpu.SemaphoreType.DMA((n_h,)),
            pltpu.SemaphoreType.DMA((n_h,)),
            pltpu.SemaphoreType.DMA((n_h,)),
            pltpu.SemaphoreType.DMA((n_h,)),
            pltpu.SemaphoreType.DMA((n_h,)),
            pltpu.SemaphoreType.DMA((n_h,)),
        ],
        compiler_params=pltpu.CompilerParams(collective_id=0),
    )(x, resid, g)


# device time: 30931 ns/iter; 1.1843x vs baseline; 1.1254x over previous
import jax
import jax.numpy as jnp
from jax import lax
from jax.experimental import pallas as pl
from jax.experimental.pallas import tpu as pltpu

N_Y = 4
M = 512
D = 512
CH = M // N_Y


def kernel(partial, resid, gamma):
    x = partial.reshape(M, D)
    g = gamma.reshape(1, D)

    def body(x_ref, resid_ref, g_ref, out_ref, recv_buf,
             rs_ss, rs_rs, ag_ss, ag_rs):
        mx = lax.axis_index("x")
        my = lax.axis_index("y")
        mz = lax.axis_index("z")

        def peer(k):
            return (mx, (my + k) % N_Y, mz)

        def rows(c):
            return pl.ds(c * CH, CH)

        barrier_sem = pltpu.get_barrier_semaphore()
        for k in (1, 2, 3):
            pl.semaphore_signal(
                barrier_sem, inc=1,
                device_id=peer(k), device_id_type=pl.DeviceIdType.MESH,
            )
        pl.semaphore_wait(barrier_sem, N_Y - 1)

        rs_rdmas = []
        for k in (1, 2, 3):
            rdma = pltpu.make_async_remote_copy(
                src_ref=x_ref.at[rows((my + k) % N_Y), :],
                dst_ref=recv_buf.at[k - 1],
                send_sem=rs_ss.at[k - 1],
                recv_sem=rs_rs.at[k - 1],
                device_id=peer(k), device_id_type=pl.DeviceIdType.MESH,
            )
            rdma.start()
            rs_rdmas.append(rdma)
        for rdma in rs_rdmas:
            rdma.wait_recv()

        y = (x_ref[rows(my), :] + recv_buf[0] + recv_buf[1] + recv_buf[2]
             + resid_ref[rows(my), :])
        rms = jnp.sqrt(jnp.mean(y * y, axis=-1, keepdims=True) + 1e-6)
        out_ref[rows(my), :] = y / rms * g_ref[:, :]

        ag_rdmas = []
        for k in (1, 2, 3):
            rdma = pltpu.make_async_remote_copy(
                src_ref=out_ref.at[rows(my), :],
                dst_ref=out_ref.at[rows(my), :],
                send_sem=ag_ss.at[k - 1],
                recv_sem=ag_rs.at[k - 1],
                device_id=peer(k), device_id_type=pl.DeviceIdType.MESH,
            )
            rdma.start()
            ag_rdmas.append(rdma)
        for rdma in ag_rdmas:
            rdma.wait_recv()
        for rdma in rs_rdmas:
            rdma.wait_send()
        for rdma in ag_rdmas:
            rdma.wait_send()

    return pl.pallas_call(
        body,
        out_shape=jax.ShapeDtypeStruct((M, D), jnp.float32),
        in_specs=[
            pl.BlockSpec(memory_space=pltpu.VMEM),
            pl.BlockSpec(memory_space=pltpu.VMEM),
            pl.BlockSpec(memory_space=pltpu.VMEM),
        ],
        out_specs=pl.BlockSpec(memory_space=pltpu.VMEM),
        scratch_shapes=[
            pltpu.VMEM((N_Y - 1, CH, D), jnp.float32),
            pltpu.SemaphoreType.DMA((N_Y - 1,)),
            pltpu.SemaphoreType.DMA((N_Y - 1,)),
            pltpu.SemaphoreType.DMA((N_Y - 1,)),
            pltpu.SemaphoreType.DMA((N_Y - 1,)),
        ],
        compiler_params=pltpu.CompilerParams(collective_id=0),
    )(x, resid, g)
